# Initial kernel scaffold; baseline (speedup 1.0000x reference)
import jax
import jax.numpy as jnp
from jax.experimental import pallas as pl


def kernel(x, edge_index, w, a):
    raise NotImplementedError("write your pallas kernel here")


# speedup vs baseline: 3.1468x; 3.1468x over previous
import functools
import math

import jax
import jax.numpy as jnp
from jax import lax
from jax.experimental import pallas as pl
from jax.experimental.pallas import tpu as pltpu
from jax.experimental.pallas import tpu_sc as plsc

NEG_SLOPE = 0.01
F = 128
FH = 64
W_ACC = 80
C = 128
N_TILES = 16
LANES = 16


def _scores_body(x_ref, w_ref, a_ref, o_ref):
    w2 = w_ref[:, 0, :]
    asrc = a_ref[:, :F] * w2
    adst = a_ref[:, F:] * w2
    mat = jnp.concatenate(
        [asrc[0:1], adst[0:1], asrc[1:2], adst[1:2],
         jnp.zeros((4, F), jnp.float32)], axis=0)
    o_ref[...] = jax.lax.dot_general(
        x_ref[...], mat, (((1,), (1,)), ((), ())),
        preferred_element_type=jnp.float32)


def _gat_sc_body(xlo_hbm, xhi_hbm, src_hbm, dst_hbm, s4_hbm, w_hbm, zeros_hbm,
                 outlo_hbm, outhi_hbm,
                 ssrc_v, sdst_v, wtab_v, sidx_v, didx_v, gbuf_v, rows_v,
                 ebuf_v, acc_sh, sem, *, n_edges, e_per_tile, n_pad):
    c = lax.axis_index("c")
    s = lax.axis_index("s")

    rpt = n_pad // N_TILES
    row_base = s * rpt

    pltpu.sync_copy(s4_hbm.at[2 * c], ssrc_v)
    pltpu.sync_copy(s4_hbm.at[2 * c + 1], sdst_v)
    pltpu.sync_copy(w_hbm.at[c], wtab_v)

    n_chunks = e_per_tile // C
    tile_base = s * e_per_tile

    for p, (x_hbm, out_hbm) in enumerate(((xlo_hbm, outlo_hbm),
                                          (xhi_hbm, outhi_hbm))):
        pltpu.sync_copy(zeros_hbm.at[pl.ds(row_base, rpt)],
                        acc_sh.at[pl.ds(row_base, rpt)])
        plsc.subcore_barrier()

        def chunk_body(ci, carry):
            base = tile_base + ci * C
            pltpu.sync_copy(src_hbm.at[pl.ds(base, C)], sidx_v)
            pltpu.sync_copy(dst_hbm.at[pl.ds(base, C)], didx_v)
            pltpu.async_copy(x_hbm.at[didx_v], gbuf_v, sem).wait()
            for k in range(C // LANES):
                isv = sidx_v[pl.ds(k * LANES, LANES)]
                idv = didx_v[pl.ds(k * LANES, LANES)]
                logit = (plsc.load_gather(ssrc_v, [isv])
                         + plsc.load_gather(sdst_v, [idv]))
                e = jnp.exp(-jnp.where(logit > 0, logit, logit * NEG_SLOPE))
                eid = base + k * LANES + lax.iota(jnp.int32, LANES)
                e = jnp.where(eid < n_edges, e, 0.0)
                ebuf_v[pl.ds(k * LANES, LANES)] = e

            def scale_row(j, carry2):
                ev = plsc.load_gather(
                    ebuf_v, [jnp.full((LANES,), j, jnp.int32)])
                for k in range(FH // LANES):
                    rows_v[j, pl.ds(k * LANES, LANES)] = (
                        gbuf_v[j, pl.ds(k * LANES, LANES)] * ev)
                rows_v[j, pl.ds(FH, LANES)] = ev
                return carry2
            lax.fori_loop(0, C, scale_row, 0, unroll=2)
            pltpu.sync_copy(rows_v, acc_sh.at[sidx_v], add=True)
            return carry

        lax.fori_loop(0, n_chunks, chunk_body, 0)
        plsc.subcore_barrier()

        off = 0
        while off < rpt:
            m = min(C, rpt - off)
            pltpu.sync_copy(acc_sh.at[pl.ds(row_base + off, m)],
                            rows_v.at[pl.ds(0, m)])

            def norm_row(j, carry2):
                den = rows_v[j, pl.ds(FH, LANES)]
                rcp = 1.0 / (den + 1e-16)
                for k in range(FH // LANES):
                    gbuf_v[j, pl.ds(k * LANES, LANES)] = (
                        rows_v[j, pl.ds(k * LANES, LANES)]
                        * wtab_v[pl.ds(p * FH + k * LANES, LANES)] * rcp)
                return carry2
            lax.fori_loop(0, m, norm_row, 0, unroll=2)
            pltpu.sync_copy(gbuf_v.at[pl.ds(0, m)],
                            out_hbm.at[c, pl.ds(row_base + off, m)])
            off += m


def kernel(x, edge_index, w, a):
    n_nodes, f_in = x.shape
    n_head = w.shape[0]
    n_edges = edge_index.shape[1]
    assert f_in == F and n_head == 2

    s_nt = pl.pallas_call(
        _scores_body,
        out_shape=jax.ShapeDtypeStruct((n_nodes, 8), jnp.float32),
    )(x, w, a)
    s4 = s_nt.T

    e_per_tile = math.ceil(n_edges / (N_TILES * C)) * C
    e_pad = e_per_tile * N_TILES
    src = edge_index[0]
    dst = edge_index[1]
    if e_pad > n_edges:
        fill = (jnp.arange(e_pad - n_edges, dtype=jnp.int32) % n_nodes)
        src = jnp.concatenate([src, fill])
        dst = jnp.concatenate([dst, fill])

    n_pad = math.ceil(n_nodes / (N_TILES * 8)) * (N_TILES * 8)
    w2d = w.reshape(n_head, F)
    zeros = jnp.zeros((n_pad, W_ACC), jnp.float32)
    xlo = x[:, :FH]
    xhi = x[:, FH:]

    mesh = plsc.VectorSubcoreMesh(core_axis_name="c", subcore_axis_name="s")
    sc_fn = pl.kernel(
        functools.partial(_gat_sc_body, n_edges=n_edges,
                          e_per_tile=e_per_tile, n_pad=n_pad),
        out_type=(
            jax.ShapeDtypeStruct((n_head, n_pad, FH), jnp.float32),
            jax.ShapeDtypeStruct((n_head, n_pad, FH), jnp.float32),
        ),
        mesh=mesh,
        compiler_params=pltpu.CompilerParams(needs_layout_passes=False,
                                             use_tc_tiling_on_sc=False),
        scratch_types=[
            pltpu.VMEM((n_nodes,), jnp.float32),
            pltpu.VMEM((n_nodes,), jnp.float32),
            pltpu.VMEM((F,), jnp.float32),
            pltpu.VMEM((C,), jnp.int32),
            pltpu.VMEM((C,), jnp.int32),
            pltpu.VMEM((C, FH), jnp.float32),
            pltpu.VMEM((C, W_ACC), jnp.float32),
            pltpu.VMEM((C,), jnp.float32),
            pltpu.VMEM_SHARED((n_pad, W_ACC), jnp.float32),
            pltpu.SemaphoreType.DMA,
        ],
    )
    out_lo, out_hi = sc_fn(x[:, :FH], x[:, FH:], src, dst, s4, w2d, zeros)
    return jnp.concatenate(
        [out_lo[:, :n_nodes, :], out_hi[:, :n_nodes, :]], axis=2)


# speedup vs baseline: 5.4272x; 1.7247x over previous
import functools
import math

import jax
import jax.numpy as jnp
from jax import lax
from jax.experimental import pallas as pl
from jax.experimental.pallas import tpu as pltpu
from jax.experimental.pallas import tpu_sc as plsc

NEG_SLOPE = 0.01
F = 128
FH = 64
W_ACC = 80
C = 128
N_TILES = 16
LANES = 16


def _scores_body(x_ref, w_ref, a_ref, o_ref):
    w2 = w_ref[:, 0, :]
    asrc = a_ref[:, :F] * w2
    adst = a_ref[:, F:] * w2
    mat = jnp.concatenate(
        [asrc[0:1], adst[0:1], asrc[1:2], adst[1:2],
         jnp.zeros((4, F), jnp.float32)], axis=0)
    o_ref[...] = jax.lax.dot_general(
        x_ref[...], mat, (((1,), (1,)), ((), ())),
        preferred_element_type=jnp.float32)


def _gat_sc_body(xlo_hbm, xhi_hbm, src_hbm, dst_hbm, s4_hbm, w_hbm, zeros_hbm,
                 outlo_hbm, outhi_hbm,
                 ssrc_v, sdst_v, wtab_v,
                 sidx0, sidx1, didx0, didx1, scur0, scur1,
                 g0, g1, r0, r1, ebuf_v, acc_sh,
                 si0, si1, sg0, sg1, ss0, ss1,
                 *, n_edges, e_per_tile, n_pad):
    sidx = (sidx0, sidx1)
    didx = (didx0, didx1)
    scur = (scur0, scur1)
    gbuf = (g0, g1)
    rows = (r0, r1)
    sem_i = (si0, si1)
    sem_g = (sg0, sg1)
    sem_s = (ss0, ss1)

    c = lax.axis_index("c")
    s = lax.axis_index("s")

    rpt = n_pad // N_TILES
    row_base = s * rpt
    n_chunks = e_per_tile // C
    tile_base = s * e_per_tile

    pltpu.sync_copy(s4_hbm.at[2 * c], ssrc_v)
    pltpu.sync_copy(s4_hbm.at[2 * c + 1], sdst_v)
    pltpu.sync_copy(w_hbm.at[c], wtab_v)

    def load_idx(ci, b):
        base = tile_base + ci * C
        pltpu.async_copy(src_hbm.at[pl.ds(base, C)], sidx[b], sem_i[b])
        pltpu.async_copy(dst_hbm.at[pl.ds(base, C)], didx[b], sem_i[b])

    def wait_idx(b):
        pltpu.make_async_copy(src_hbm.at[pl.ds(0, C)], sidx[b],
                              sem_i[b]).wait()
        pltpu.make_async_copy(dst_hbm.at[pl.ds(0, C)], didx[b],
                              sem_i[b]).wait()

    for p, (x_hbm, out_hbm) in enumerate(((xlo_hbm, outlo_hbm),
                                          (xhi_hbm, outhi_hbm))):
        pltpu.sync_copy(zeros_hbm.at[pl.ds(row_base, rpt)],
                        acc_sh.at[pl.ds(row_base, rpt)])
        plsc.subcore_barrier()

        def process(ci, b, wait_scatter):
            if wait_scatter:
                pltpu.make_async_copy(rows[b], acc_sh.at[scur[b]],
                                      sem_s[b]).wait()
            for k in range(C // LANES):
                isv = sidx[b][pl.ds(k * LANES, LANES)]
                idv = didx[b][pl.ds(k * LANES, LANES)]
                logit = (plsc.load_gather(ssrc_v, [isv])
                         + plsc.load_gather(sdst_v, [idv]))
                e = jnp.exp(-jnp.where(logit > 0, logit, logit * NEG_SLOPE))
                eid = tile_base + ci * C + k * LANES + lax.iota(jnp.int32,
                                                                LANES)
                e = jnp.where(eid < n_edges, e, 0.0)
                ebuf_v[pl.ds(k * LANES, LANES)] = e
                scur[b][pl.ds(k * LANES, LANES)] = isv
            load_idx(jnp.minimum(ci + 2, n_chunks - 1), b)
            wait_idx(1 - b)
            pltpu.async_copy(x_hbm.at[didx[1 - b]], gbuf[1 - b],
                             sem_g[1 - b])
            pltpu.make_async_copy(x_hbm.at[didx[b]], gbuf[b],
                                  sem_g[b]).wait()

            def scale_row(j, carry2):
                ev = plsc.load_gather(
                    ebuf_v, [jnp.full((LANES,), j, jnp.int32)])
                for k in range(FH // LANES):
                    rows[b][j, pl.ds(k * LANES, LANES)] = (
                        gbuf[b][j, pl.ds(k * LANES, LANES)] * ev)
                rows[b][j, pl.ds(FH, LANES)] = ev
                return carry2
            lax.fori_loop(0, C, scale_row, 0, unroll=4)
            pltpu.async_copy(rows[b], acc_sh.at[scur[b]], sem_s[b], add=True)

        load_idx(0, 0)
        wait_idx(0)
        load_idx(1, 1)
        pltpu.async_copy(x_hbm.at[didx[0]], gbuf[0], sem_g[0])
        process(0, 0, wait_scatter=False)
        process(1, 1, wait_scatter=False)

        def pair_body(i2, carry):
            process(2 * i2, 0, wait_scatter=True)
            process(2 * i2 + 1, 1, wait_scatter=True)
            return carry
        lax.fori_loop(1, n_chunks // 2, pair_body, 0)

        pltpu.make_async_copy(rows[0], acc_sh.at[scur[0]], sem_s[0]).wait()
        pltpu.make_async_copy(rows[1], acc_sh.at[scur[1]], sem_s[1]).wait()
        pltpu.make_async_copy(x_hbm.at[didx[0]], gbuf[0], sem_g[0]).wait()
        wait_idx(1)
        plsc.subcore_barrier()

        off = 0
        while off < rpt:
            m = min(C, rpt - off)
            pltpu.sync_copy(acc_sh.at[pl.ds(row_base + off, m)],
                            r0.at[pl.ds(0, m)])

            def norm_row(j, carry2):
                den = r0[j, pl.ds(FH, LANES)]
                rcp = 1.0 / (den + 1e-16)
                for k in range(FH // LANES):
                    g0[j, pl.ds(k * LANES, LANES)] = (
                        r0[j, pl.ds(k * LANES, LANES)]
                        * wtab_v[pl.ds(p * FH + k * LANES, LANES)] * rcp)
                return carry2
            lax.fori_loop(0, m, norm_row, 0, unroll=4)
            pltpu.sync_copy(g0.at[pl.ds(0, m)],
                            out_hbm.at[c, pl.ds(row_base + off, m)])
            off += m


def kernel(x, edge_index, w, a):
    n_nodes, f_in = x.shape
    n_head = w.shape[0]
    n_edges = edge_index.shape[1]
    assert f_in == F and n_head == 2

    s_nt = pl.pallas_call(
        _scores_body,
        out_shape=jax.ShapeDtypeStruct((n_nodes, 8), jnp.float32),
    )(x, w, a)
    s4 = s_nt.T

    e_per_tile = math.ceil(n_edges / (N_TILES * 2 * C)) * 2 * C
    e_pad = e_per_tile * N_TILES
    src = edge_index[0]
    dst = edge_index[1]
    if e_pad > n_edges:
        fill = (jnp.arange(e_pad - n_edges, dtype=jnp.int32) % n_nodes)
        src = jnp.concatenate([src, fill])
        dst = jnp.concatenate([dst, fill])

    n_pad = math.ceil(n_nodes / (N_TILES * 8)) * (N_TILES * 8)
    w2d = w.reshape(n_head, F)
    zeros = jnp.zeros((n_pad, W_ACC), jnp.float32)

    mesh = plsc.VectorSubcoreMesh(core_axis_name="c", subcore_axis_name="s")
    sc_fn = pl.kernel(
        functools.partial(_gat_sc_body, n_edges=n_edges,
                          e_per_tile=e_per_tile, n_pad=n_pad),
        out_type=(
            jax.ShapeDtypeStruct((n_head, n_pad, FH), jnp.float32),
            jax.ShapeDtypeStruct((n_head, n_pad, FH), jnp.float32),
        ),
        mesh=mesh,
        compiler_params=pltpu.CompilerParams(needs_layout_passes=False,
                                             use_tc_tiling_on_sc=False),
        scratch_types=[
            pltpu.VMEM((n_nodes,), jnp.float32),
            pltpu.VMEM((n_nodes,), jnp.float32),
            pltpu.VMEM((F,), jnp.float32),
            pltpu.VMEM((C,), jnp.int32),
            pltpu.VMEM((C,), jnp.int32),
            pltpu.VMEM((C,), jnp.int32),
            pltpu.VMEM((C,), jnp.int32),
            pltpu.VMEM((C,), jnp.int32),
            pltpu.VMEM((C,), jnp.int32),
            pltpu.VMEM((C, FH), jnp.float32),
            pltpu.VMEM((C, FH), jnp.float32),
            pltpu.VMEM((C, W_ACC), jnp.float32),
            pltpu.VMEM((C, W_ACC), jnp.float32),
            pltpu.VMEM((C,), jnp.float32),
            pltpu.VMEM_SHARED((n_pad, W_ACC), jnp.float32),
            pltpu.SemaphoreType.DMA,
            pltpu.SemaphoreType.DMA,
            pltpu.SemaphoreType.DMA,
            pltpu.SemaphoreType.DMA,
            pltpu.SemaphoreType.DMA,
            pltpu.SemaphoreType.DMA,
        ],
    )
    out_lo, out_hi = sc_fn(x[:, :FH], x[:, FH:], src, dst, s4, w2d, zeros)
    return jnp.concatenate(
        [out_lo[:, :n_nodes, :], out_hi[:, :n_nodes, :]], axis=2)
